# baseline (speedup 1.0000x reference)
import jax
import jax.numpy as jnp
from jax import lax
from jax.experimental import pallas as pl
from jax.experimental.pallas import tpu as pltpu
from jax.experimental.pallas import tpu_sc as plsc

N = 1048576
D = 32
M_COORD = 4194304
PAD = 8192

NC = 2
NS = 16
NW = NC * NS

NB = 64
BSPAN = M_COORD // NB
CAP = 768
CHUNK = N // NW
_SCB = 1

_SC_PARAMS = pltpu.CompilerParams(
    use_tc_tiling_on_sc=False, needs_layout_passes=False)
_MESH = dict(mesh=plsc.VectorSubcoreMesh(core_axis_name="c", subcore_axis_name="s"))

_K1A_H = 16384


def _k1a_body(gi_ref, pc_ref, pi_ref, cnt_ref, idx_v, stage_c, stage_i, off_tbl):
    wid = lax.axis_index("s") * NC + lax.axis_index("c")

    @pl.loop(0, NB // 16)
    def _(z):
        off_tbl[pl.ds(16 * z, 16)] = jnp.zeros((16,), jnp.int32)

    zero16 = jnp.zeros((16,), jnp.int32)

    @pl.loop(0, CHUNK // _K1A_H)
    def _(h):
        pltpu.sync_copy(gi_ref.at[pl.ds(wid, 1), pl.ds(h * _K1A_H, _K1A_H)], idx_v)
        base_i = wid * CHUNK + h * _K1A_H + 1

        @pl.loop(0, _K1A_H // 16)
        def _(k):
            c = idx_v[0, pl.ds(16 * k, 16)]
            iv = base_i + 16 * k + lax.iota(jnp.int32, 16)
            b = lax.shift_right_logical(c, 16)
            rank, last_m = plsc.scan_count(b)
            off = plsc.load_gather(off_tbl, [b])
            pos = off + rank - _SCB
            okm = pos < CAP
            plsc.store_scatter(stage_c, [b, zero16, pos], c, mask=okm)
            plsc.store_scatter(stage_i, [b, zero16, pos], iv, mask=okm)
            plsc.store_scatter(off_tbl, [b], jnp.minimum(pos + 1, CAP), mask=last_m)

    pltpu.sync_copy(stage_c, pc_ref.at[:, pl.ds(wid, 1), :])
    pltpu.sync_copy(stage_i, pi_ref.at[:, pl.ds(wid, 1), :])
    pltpu.sync_copy(off_tbl, cnt_ref.at[pl.ds(wid * NB, NB)])


def _k1a(gi2d):
    kern = pl.kernel(
        _k1a_body,
        out_type=(
            jax.ShapeDtypeStruct((NB, NW, CAP), jnp.int32),
            jax.ShapeDtypeStruct((NB, NW, CAP), jnp.int32),
            jax.ShapeDtypeStruct((NW * NB,), jnp.int32),
        ),
        compiler_params=_SC_PARAMS,
        scratch_types=[
            pltpu.VMEM((1, _K1A_H), jnp.int32),
            pltpu.VMEM((NB, 1, CAP), jnp.int32),
            pltpu.VMEM((NB, 1, CAP), jnp.int32),
            pltpu.VMEM((NB,), jnp.int32),
        ],
        **_MESH,
    )
    return kern(gi2d)


def _k1b_body(pc_ref, pi_ref, cnt_ref, table_ref, pcv, piv, cnts_v, tbl_v):
    wid = lax.axis_index("s") * NC + lax.axis_index("c")
    pltpu.sync_copy(cnt_ref, cnts_v)

    @pl.loop(0, NB // NW)
    def _(bk):
        b = wid * (NB // NW) + bk

        @pl.loop(0, BSPAN // 16)
        def _(z):
            tbl_v[pl.ds(16 * z, 16)] = jnp.zeros((16,), jnp.int32)

        pltpu.sync_copy(pc_ref.at[pl.ds(b, 1), :, :], pcv)
        pltpu.sync_copy(pi_ref.at[pl.ds(b, 1), :, :], piv)

        @pl.loop(0, NW)
        def _(s):
            cnt_idx = jnp.broadcast_to(s * NB + b, (16,)).astype(jnp.int32)
            cnt = plsc.load_gather(cnts_v, [cnt_idx])

            @pl.loop(0, CAP // 16)
            def _(k):
                lane = 16 * k + lax.iota(jnp.int32, 16)
                m = lane < cnt
                c = pcv[0, s, pl.ds(16 * k, 16)]
                iv = piv[0, s, pl.ds(16 * k, 16)]
                lc = c & (BSPAN - 1)
                _, lastc = plsc.scan_count(lc, mask=m)
                plsc.store_scatter(tbl_v, [lc], iv, mask=m & lastc)

        pltpu.sync_copy(tbl_v, table_ref.at[pl.ds(b * BSPAN, BSPAN)])


def _k1b(pc, pi, cnts):
    kern = pl.kernel(
        _k1b_body,
        out_type=jax.ShapeDtypeStruct((M_COORD,), jnp.int32),
        compiler_params=_SC_PARAMS,
        scratch_types=[
            pltpu.VMEM((1, NW, CAP), jnp.int32),
            pltpu.VMEM((1, NW, CAP), jnp.int32),
            pltpu.VMEM((NW * NB,), jnp.int32),
            pltpu.VMEM((BSPAN,), jnp.int32),
        ],
        **_MESH,
    )
    return kern(pc, pi, cnts)


K2_CHUNK = 1024
K2_STEPS = N // (NW * K2_CHUNK)


def _k2_body(oi_ref, table_ref, dext_ref, out_ref,
             oidx_v, pv_v, gidx_v, dense_v, wsem):
    wid = lax.axis_index("s") * NC + lax.axis_index("c")
    wbase = wid * K2_STEPS * K2_CHUNK

    zero16f = jnp.zeros((16,), jnp.float32)

    def chunk(b, u, first):
        jbase = wbase + b * K2_CHUNK
        pltpu.sync_copy(oi_ref.at[pl.ds(jbase, K2_CHUNK)], oidx_v.at[u])
        pltpu.sync_copy(table_ref.at[oidx_v.at[u]], pv_v.at[u])

        @pl.loop(0, K2_CHUNK // 16)
        def _(k):
            pv = pv_v[u, pl.ds(16 * k, 16)]
            gidx_v[u, pl.ds(16 * k, 16)] = jnp.maximum(pv - 1, 0)

        if not first:
            pltpu.make_async_copy(
                dense_v.at[u],
                out_ref.at[pl.ds(wbase, K2_CHUNK), :], wsem.at[u]).wait()
        pltpu.sync_copy(dext_ref.at[gidx_v.at[u]], dense_v.at[u])

        u16 = jnp.full((16,), u, jnp.int32)

        @pl.loop(0, K2_CHUNK // 16)
        def _(k):
            inv = pv_v[u, pl.ds(16 * k, 16)] < 1
            rows = 16 * k + lax.iota(jnp.int32, 16)
            for h in range(D):
                plsc.store_scatter(
                    dense_v, [u16, rows, jnp.full((16,), h, jnp.int32)],
                    zero16f, mask=inv)

        pltpu.async_copy(dense_v.at[u],
                         out_ref.at[pl.ds(jbase, K2_CHUNK), :], wsem.at[u])

    for u in (0, 1):
        chunk(u, u, True)

    @pl.loop(2, K2_STEPS, step=2)
    def _(g):
        for u in (0, 1):
            chunk(g + u, u, False)

    for u in (0, 1):
        pltpu.make_async_copy(
            dense_v.at[u],
            out_ref.at[pl.ds(wbase, K2_CHUNK), :], wsem.at[u]).wait()


def _k2(oi32, table, data_ext):
    kern = pl.kernel(
        _k2_body,
        out_type=jax.ShapeDtypeStruct((N, D), jnp.float32),
        compiler_params=_SC_PARAMS,
        scratch_types=[
            pltpu.VMEM((2, K2_CHUNK), jnp.int32),
            pltpu.VMEM((2, K2_CHUNK), jnp.int32),
            pltpu.VMEM((2, K2_CHUNK), jnp.int32),
            pltpu.VMEM((2, K2_CHUNK, D), jnp.float32),
            pltpu.SemaphoreType.DMA((2,)),
        ],
        **_MESH,
    )
    return kern(oi32, table, data_ext)


@jax.jit
def kernel(data, grid_idx, other_grid_idx):
    gi2d = grid_idx.astype(jnp.int32).reshape(NW, CHUNK)
    oi32 = other_grid_idx.astype(jnp.int32)
    pc, pi, cnts = _k1a(gi2d)
    table = _k1b(pc, pi, cnts)
    return _k2(oi32, table, data)


# speedup vs baseline: 6.6487x; 6.6487x over previous
import jax
import jax.numpy as jnp
from jax import lax
from jax.experimental import pallas as pl
from jax.experimental.pallas import tpu as pltpu
from jax.experimental.pallas import tpu_sc as plsc

N = 1048576
D = 32
M_COORD = 4194304
PAD = 8192

NC = 2
NS = 16
NW = NC * NS

NB = 64
BSPAN = M_COORD // NB
CAP = 768
CHUNK = N // NW
_SCB = 1

_SC_PARAMS = pltpu.CompilerParams(
    use_tc_tiling_on_sc=False, needs_layout_passes=False)
_MESH = dict(mesh=plsc.VectorSubcoreMesh(core_axis_name="c", subcore_axis_name="s"))

_EXT_BLK = 262144
_NE = N * D
_PADE = PAD * D


def _ext_body(x_ref, o_ref):
    i = pl.program_id(0)
    o_ref[...] = jnp.where(i < _NE // _EXT_BLK, x_ref[...], 0.0)


def _build_data_ext(data):
    x = data.reshape(_NE)
    nblk = (_NE + _PADE) // _EXT_BLK
    out = pl.pallas_call(
        _ext_body,
        grid=(nblk,),
        in_specs=[pl.BlockSpec((_EXT_BLK,),
                               lambda i: (jnp.minimum(i, _NE // _EXT_BLK - 1),))],
        out_specs=pl.BlockSpec((_EXT_BLK,), lambda i: (i,)),
        out_shape=jax.ShapeDtypeStruct((_NE + _PADE,), jnp.float32),
    )(x)
    return out.reshape(N + PAD, D)


_K1A_H = 16384


def _k1a_body(gi_ref, pc_ref, pi_ref, cnt_ref, idx_v, stage_c, stage_i, off_tbl):
    wid = lax.axis_index("s") * NC + lax.axis_index("c")

    @pl.loop(0, NB // 16)
    def _(z):
        off_tbl[pl.ds(16 * z, 16)] = jnp.zeros((16,), jnp.int32)

    zero16 = jnp.zeros((16,), jnp.int32)

    @pl.loop(0, CHUNK // _K1A_H)
    def _(h):
        pltpu.sync_copy(gi_ref.at[pl.ds(wid, 1), pl.ds(h * _K1A_H, _K1A_H)], idx_v)
        base_i = wid * CHUNK + h * _K1A_H + 1

        @pl.loop(0, _K1A_H // 16)
        def _(k):
            c = idx_v[0, pl.ds(16 * k, 16)]
            iv = base_i + 16 * k + lax.iota(jnp.int32, 16)
            b = lax.shift_right_logical(c, 16)
            rank, last_m = plsc.scan_count(b)
            off = plsc.load_gather(off_tbl, [b])
            pos = off + rank - _SCB
            okm = pos < CAP
            plsc.store_scatter(stage_c, [b, zero16, pos], c, mask=okm)
            plsc.store_scatter(stage_i, [b, zero16, pos], iv, mask=okm)
            plsc.store_scatter(off_tbl, [b], jnp.minimum(pos + 1, CAP), mask=last_m)

    pltpu.sync_copy(stage_c, pc_ref.at[:, pl.ds(wid, 1), :])
    pltpu.sync_copy(stage_i, pi_ref.at[:, pl.ds(wid, 1), :])
    pltpu.sync_copy(off_tbl, cnt_ref.at[pl.ds(wid * NB, NB)])


def _k1a(gi2d):
    kern = pl.kernel(
        _k1a_body,
        out_type=(
            jax.ShapeDtypeStruct((NB, NW, CAP), jnp.int32),
            jax.ShapeDtypeStruct((NB, NW, CAP), jnp.int32),
            jax.ShapeDtypeStruct((NW * NB,), jnp.int32),
        ),
        compiler_params=_SC_PARAMS,
        scratch_types=[
            pltpu.VMEM((1, _K1A_H), jnp.int32),
            pltpu.VMEM((NB, 1, CAP), jnp.int32),
            pltpu.VMEM((NB, 1, CAP), jnp.int32),
            pltpu.VMEM((NB,), jnp.int32),
        ],
        **_MESH,
    )
    return kern(gi2d)


def _k1b_body(pc_ref, pi_ref, cnt_ref, table_ref, pcv, piv, cnts_v, tbl_v):
    wid = lax.axis_index("s") * NC + lax.axis_index("c")
    pltpu.sync_copy(cnt_ref, cnts_v)

    @pl.loop(0, NB // NW)
    def _(bk):
        b = wid * (NB // NW) + bk

        @pl.loop(0, BSPAN // 16)
        def _(z):
            tbl_v[pl.ds(16 * z, 16)] = jnp.zeros((16,), jnp.int32)

        pltpu.sync_copy(pc_ref.at[pl.ds(b, 1), :, :], pcv)
        pltpu.sync_copy(pi_ref.at[pl.ds(b, 1), :, :], piv)

        @pl.loop(0, NW)
        def _(s):
            cnt_idx = jnp.broadcast_to(s * NB + b, (16,)).astype(jnp.int32)
            cnt = plsc.load_gather(cnts_v, [cnt_idx])

            @pl.loop(0, CAP // 16)
            def _(k):
                lane = 16 * k + lax.iota(jnp.int32, 16)
                m = lane < cnt
                c = pcv[0, s, pl.ds(16 * k, 16)]
                iv = piv[0, s, pl.ds(16 * k, 16)]
                lc = c & (BSPAN - 1)
                _, lastc = plsc.scan_count(lc, mask=m)
                plsc.store_scatter(tbl_v, [lc], iv, mask=m & lastc)

        pltpu.sync_copy(tbl_v, table_ref.at[pl.ds(b * BSPAN, BSPAN)])


def _k1b(pc, pi, cnts):
    kern = pl.kernel(
        _k1b_body,
        out_type=jax.ShapeDtypeStruct((M_COORD,), jnp.int32),
        compiler_params=_SC_PARAMS,
        scratch_types=[
            pltpu.VMEM((1, NW, CAP), jnp.int32),
            pltpu.VMEM((1, NW, CAP), jnp.int32),
            pltpu.VMEM((NW * NB,), jnp.int32),
            pltpu.VMEM((BSPAN,), jnp.int32),
        ],
        **_MESH,
    )
    return kern(pc, pi, cnts)


K2_CHUNK = 1024
K2_STEPS = N // (NW * K2_CHUNK)


def _k2_body(oi_ref, table_ref, dext_ref, out_ref,
             oidx_v, pv_v, gidx_v, dense_v, wsem):
    wid = lax.axis_index("s") * NC + lax.axis_index("c")
    wbase = wid * K2_STEPS * K2_CHUNK

    zero16f = jnp.zeros((16,), jnp.float32)

    def chunk(b, u, first):
        jbase = wbase + b * K2_CHUNK
        pltpu.sync_copy(oi_ref.at[pl.ds(jbase, K2_CHUNK)], oidx_v.at[u])
        pltpu.sync_copy(table_ref.at[oidx_v.at[u]], pv_v.at[u])

        @pl.loop(0, K2_CHUNK // 16)
        def _(k):
            pv = pv_v[u, pl.ds(16 * k, 16)]
            j16 = jbase + 16 * k + lax.iota(jnp.int32, 16)
            gidx_v[u, pl.ds(16 * k, 16)] = jnp.where(
                pv > 0, pv - 1, N + (j16 & (PAD - 1)))

        if not first:
            pltpu.make_async_copy(
                dense_v.at[u],
                out_ref.at[pl.ds(wbase, K2_CHUNK), :], wsem.at[u]).wait()
        pltpu.sync_copy(dext_ref.at[gidx_v.at[u]], dense_v.at[u])
        pltpu.async_copy(dense_v.at[u],
                         out_ref.at[pl.ds(jbase, K2_CHUNK), :], wsem.at[u])

    for u in (0, 1):
        chunk(u, u, True)

    @pl.loop(2, K2_STEPS, step=2)
    def _(g):
        for u in (0, 1):
            chunk(g + u, u, False)

    for u in (0, 1):
        pltpu.make_async_copy(
            dense_v.at[u],
            out_ref.at[pl.ds(wbase, K2_CHUNK), :], wsem.at[u]).wait()


def _k2(oi32, table, data_ext):
    kern = pl.kernel(
        _k2_body,
        out_type=jax.ShapeDtypeStruct((N, D), jnp.float32),
        compiler_params=_SC_PARAMS,
        scratch_types=[
            pltpu.VMEM((2, K2_CHUNK), jnp.int32),
            pltpu.VMEM((2, K2_CHUNK), jnp.int32),
            pltpu.VMEM((2, K2_CHUNK), jnp.int32),
            pltpu.VMEM((2, K2_CHUNK, D), jnp.float32),
            pltpu.SemaphoreType.DMA((2,)),
        ],
        **_MESH,
    )
    return kern(oi32, table, data_ext)


@jax.jit
def kernel(data, grid_idx, other_grid_idx):
    gi2d = grid_idx.astype(jnp.int32).reshape(NW, CHUNK)
    oi32 = other_grid_idx.astype(jnp.int32)
    data_ext = _build_data_ext(data)
    pc, pi, cnts = _k1a(gi2d)
    table = _k1b(pc, pi, cnts)
    return _k2(oi32, table, data_ext)


# speedup vs baseline: 6.7721x; 1.0186x over previous
import jax
import jax.numpy as jnp
from jax import lax
from jax.experimental import pallas as pl
from jax.experimental.pallas import tpu as pltpu
from jax.experimental.pallas import tpu_sc as plsc

N = 1048576
D = 32
M_COORD = 4194304
PAD = 8192

NC = 2
NS = 16
NW = NC * NS

NB = 64
BSPAN = M_COORD // NB
CAP = 768
CHUNK = N // NW
_SCB = 1

_SC_PARAMS = pltpu.CompilerParams(
    use_tc_tiling_on_sc=False, needs_layout_passes=False)
_MESH = dict(mesh=plsc.VectorSubcoreMesh(core_axis_name="c", subcore_axis_name="s"))

_EXT_BLK = 262144
_NE = N * D
_PADE = PAD * D


def _ext_body(x_ref, o_ref):
    i = pl.program_id(0)
    o_ref[...] = jnp.where(i < _NE // _EXT_BLK, x_ref[...], 0.0)


def _build_data_ext(data):
    x = data.reshape(_NE)
    nblk = (_NE + _PADE) // _EXT_BLK
    out = pl.pallas_call(
        _ext_body,
        grid=(nblk,),
        in_specs=[pl.BlockSpec((_EXT_BLK,),
                               lambda i: (jnp.minimum(i, _NE // _EXT_BLK - 1),))],
        out_specs=pl.BlockSpec((_EXT_BLK,), lambda i: (i,)),
        out_shape=jax.ShapeDtypeStruct((_NE + _PADE,), jnp.float32),
    )(x)
    return out.reshape(N + PAD, D)


_K1A_H = 16384


def _k1a_body(gi_ref, pc_ref, pi_ref, cnt_ref, idx_v, stage_c, stage_i, off_tbl):
    wid = lax.axis_index("s") * NC + lax.axis_index("c")

    @pl.loop(0, NB // 16)
    def _(z):
        off_tbl[pl.ds(16 * z, 16)] = jnp.zeros((16,), jnp.int32)

    zero16 = jnp.zeros((16,), jnp.int32)

    @pl.loop(0, CHUNK // _K1A_H)
    def _(h):
        pltpu.sync_copy(gi_ref.at[pl.ds(wid, 1), pl.ds(h * _K1A_H, _K1A_H)], idx_v)
        base_i = wid * CHUNK + h * _K1A_H + 1

        @pl.loop(0, _K1A_H // 16)
        def _(k):
            c = idx_v[0, pl.ds(16 * k, 16)]
            iv = base_i + 16 * k + lax.iota(jnp.int32, 16)
            b = lax.shift_right_logical(c, 16)
            rank, last_m = plsc.scan_count(b)
            off = plsc.load_gather(off_tbl, [b])
            pos = off + rank - _SCB
            okm = pos < CAP
            plsc.store_scatter(stage_c, [b, zero16, pos], c, mask=okm)
            plsc.store_scatter(stage_i, [b, zero16, pos], iv, mask=okm)
            plsc.store_scatter(off_tbl, [b], jnp.minimum(pos + 1, CAP), mask=last_m)

    pltpu.sync_copy(stage_c, pc_ref.at[:, pl.ds(wid, 1), :])
    pltpu.sync_copy(stage_i, pi_ref.at[:, pl.ds(wid, 1), :])
    pltpu.sync_copy(off_tbl, cnt_ref.at[pl.ds(wid * NB, NB)])


def _k1a(gi2d):
    kern = pl.kernel(
        _k1a_body,
        out_type=(
            jax.ShapeDtypeStruct((NB, NW, CAP), jnp.int32),
            jax.ShapeDtypeStruct((NB, NW, CAP), jnp.int32),
            jax.ShapeDtypeStruct((NW * NB,), jnp.int32),
        ),
        compiler_params=_SC_PARAMS,
        scratch_types=[
            pltpu.VMEM((1, _K1A_H), jnp.int32),
            pltpu.VMEM((NB, 1, CAP), jnp.int32),
            pltpu.VMEM((NB, 1, CAP), jnp.int32),
            pltpu.VMEM((NB,), jnp.int32),
        ],
        **_MESH,
    )
    return kern(gi2d)


def _k1b_body(pc_ref, pi_ref, cnt_ref, table_ref, pcv, piv, cnts_v, tbl_v):
    wid = lax.axis_index("s") * NC + lax.axis_index("c")
    pltpu.sync_copy(cnt_ref, cnts_v)

    @pl.loop(0, NB // NW)
    def _(bk):
        b = wid * (NB // NW) + bk

        @pl.loop(0, BSPAN // 16)
        def _(z):
            tbl_v[pl.ds(16 * z, 16)] = jnp.zeros((16,), jnp.int32)

        pltpu.sync_copy(pc_ref.at[pl.ds(b, 1), :, :], pcv)
        pltpu.sync_copy(pi_ref.at[pl.ds(b, 1), :, :], piv)

        @pl.loop(0, NW)
        def _(s):
            cnt_idx = jnp.broadcast_to(s * NB + b, (16,)).astype(jnp.int32)
            cnt = plsc.load_gather(cnts_v, [cnt_idx])
            kmax = (jnp.max(cnt) + 15) >> 4

            @pl.loop(0, kmax)
            def _(k):
                lane = 16 * k + lax.iota(jnp.int32, 16)
                m = lane < cnt
                c = pcv[0, s, pl.ds(16 * k, 16)]
                iv = piv[0, s, pl.ds(16 * k, 16)]
                lc = c & (BSPAN - 1)
                _, lastc = plsc.scan_count(lc, mask=m)
                plsc.store_scatter(tbl_v, [lc], iv, mask=m & lastc)

        pltpu.sync_copy(tbl_v, table_ref.at[pl.ds(b * BSPAN, BSPAN)])


def _k1b(pc, pi, cnts):
    kern = pl.kernel(
        _k1b_body,
        out_type=jax.ShapeDtypeStruct((M_COORD,), jnp.int32),
        compiler_params=_SC_PARAMS,
        scratch_types=[
            pltpu.VMEM((1, NW, CAP), jnp.int32),
            pltpu.VMEM((1, NW, CAP), jnp.int32),
            pltpu.VMEM((NW * NB,), jnp.int32),
            pltpu.VMEM((BSPAN,), jnp.int32),
        ],
        **_MESH,
    )
    return kern(pc, pi, cnts)


K2_CHUNK = 1024
K2_STEPS = N // (NW * K2_CHUNK)


def _k2_body(oi_ref, table_ref, dext_ref, out_ref,
             oidx_v, pv_v, gidx_v, dense_v, gsem, wsem):
    wid = lax.axis_index("s") * NC + lax.axis_index("c")
    wbase = wid * K2_STEPS * K2_CHUNK

    def chunk(g, u, first, write_prev):
        b = g + u
        jbase = wbase + b * K2_CHUNK
        pltpu.sync_copy(oi_ref.at[pl.ds(jbase, K2_CHUNK)], oidx_v.at[u])
        pltpu.sync_copy(table_ref.at[oidx_v.at[u]], pv_v.at[u])

        @pl.loop(0, K2_CHUNK // 16)
        def _(k):
            pv = pv_v[u, pl.ds(16 * k, 16)]
            j16 = jbase + 16 * k + lax.iota(jnp.int32, 16)
            gidx_v[u, pl.ds(16 * k, 16)] = jnp.where(
                pv > 0, pv - 1, N + (j16 & (PAD - 1)))

        if not first:
            pltpu.make_async_copy(
                dense_v.at[u],
                out_ref.at[pl.ds(wbase, K2_CHUNK), :], wsem.at[u]).wait()
        pltpu.async_copy(dext_ref.at[gidx_v.at[u]], dense_v.at[u], gsem.at[u])
        if write_prev:
            pltpu.make_async_copy(
                dext_ref.at[gidx_v.at[1 - u]], dense_v.at[1 - u],
                gsem.at[1 - u]).wait()
            pltpu.async_copy(
                dense_v.at[1 - u],
                out_ref.at[pl.ds(jbase - K2_CHUNK, K2_CHUNK), :],
                wsem.at[1 - u])

    chunk(0, 0, True, False)
    chunk(0, 1, True, True)

    @pl.loop(2, K2_STEPS, step=2)
    def _(g):
        chunk(g, 0, False, True)
        chunk(g, 1, False, True)

    last = K2_STEPS - 1
    pltpu.make_async_copy(
        dext_ref.at[gidx_v.at[1]], dense_v.at[1], gsem.at[1]).wait()
    pltpu.sync_copy(dense_v.at[1],
                    out_ref.at[pl.ds(wbase + last * K2_CHUNK, K2_CHUNK), :])
    pltpu.make_async_copy(
        dense_v.at[0],
        out_ref.at[pl.ds(wbase, K2_CHUNK), :], wsem.at[0]).wait()


def _k2(oi32, table, data_ext):
    kern = pl.kernel(
        _k2_body,
        out_type=jax.ShapeDtypeStruct((N, D), jnp.float32),
        compiler_params=_SC_PARAMS,
        scratch_types=[
            pltpu.VMEM((2, K2_CHUNK), jnp.int32),
            pltpu.VMEM((2, K2_CHUNK), jnp.int32),
            pltpu.VMEM((2, K2_CHUNK), jnp.int32),
            pltpu.VMEM((2, K2_CHUNK, D), jnp.float32),
            pltpu.SemaphoreType.DMA((2,)),
            pltpu.SemaphoreType.DMA((2,)),
        ],
        **_MESH,
    )
    return kern(oi32, table, data_ext)


@jax.jit
def kernel(data, grid_idx, other_grid_idx):
    gi2d = grid_idx.astype(jnp.int32).reshape(NW, CHUNK)
    oi32 = other_grid_idx.astype(jnp.int32)
    data_ext = _build_data_ext(data)
    pc, pi, cnts = _k1a(gi2d)
    table = _k1b(pc, pi, cnts)
    return _k2(oi32, table, data_ext)
